# Initial kernel scaffold; baseline (speedup 1.0000x reference)
import jax
import jax.numpy as jnp
from jax.experimental import pallas as pl


def kernel(x, W, b, gamma, beta):
    raise NotImplementedError("write your pallas kernel here")


# speedup vs baseline: 11.3450x; 11.3450x over previous
import functools

import jax
import jax.numpy as jnp
from jax import lax
from jax.experimental import pallas as pl
from jax.experimental.pallas import tpu as pltpu
from jax.experimental.pallas import tpu_sc as plsc

B, C, N, K, O = 8, 64, 2048, 16, 64
RB = 256
NB = N // RB
NC, NS = 2, 16
NW = NC * NS
PW = (B * N) // NW
CP = 8
NCHUNK = PW // CP
GP = 128


def _knn_body(x_ref, w_ref, bias_ref, idx_ref, c_ref, g_ref, d_ref):
    b = pl.program_id(0)
    rb = pl.program_id(1)
    x = x_ref[0]
    xrow = x_ref[0, :, pl.ds(rb * RB, RB)]

    gram = lax.dot_general(xrow, x, (((0,), (0,)), ((), ())),
                           preferred_element_type=jnp.float32)
    sqcol = jnp.sum(x * x, axis=0, keepdims=True)
    d_ref[...] = sqcol - 2.0 * gram

    iota_col = lax.broadcasted_iota(jnp.int32, (RB, N), 1)
    lane_k = lax.broadcasted_iota(jnp.int32, (RB, K), 1)

    def body(k, mids):
        d = d_ref[...]
        mval = jnp.min(d, axis=1, keepdims=True)
        cand = jnp.where(d <= mval, iota_col, N)
        midx = jnp.min(cand, axis=1, keepdims=True)
        mids = jnp.where(lane_k == k, midx + b * N, mids)
        d_ref[...] = jnp.where(iota_col == midx, jnp.inf, d)
        return mids

    mids = lax.fori_loop(0, K, body, jnp.zeros((RB, K), jnp.int32))
    idx_ref[...] = mids

    w = w_ref[...]
    wd = w[:, :C] - w[:, C:]
    w2 = w[:, C:]
    c_ref[...] = lax.dot_general(xrow, wd, (((0,), (1,)), ((), ())),
                                 preferred_element_type=jnp.float32) + bias_ref[...]
    g = lax.dot_general(xrow, w2, (((0,), (1,)), ((), ())),
                        preferred_element_type=jnp.float32)
    g_ref[...] = jnp.concatenate([g, jnp.zeros((RB, GP - O), jnp.float32)], axis=1)


def _knn_call(x, w, bias2d, interpret=False):
    return pl.pallas_call(
        _knn_body,
        grid=(B, NB),
        in_specs=[
            pl.BlockSpec((1, C, N), lambda b, rb: (b, 0, 0)),
            pl.BlockSpec((O, 2 * C), lambda b, rb: (0, 0)),
            pl.BlockSpec((1, O), lambda b, rb: (0, 0)),
        ],
        out_specs=[
            pl.BlockSpec((RB, K), lambda b, rb: (b * NB + rb, 0)),
            pl.BlockSpec((RB, O), lambda b, rb: (b * NB + rb, 0)),
            pl.BlockSpec((RB, GP), lambda b, rb: (b * NB + rb, 0)),
        ],
        out_shape=[
            jax.ShapeDtypeStruct((B * N, K), jnp.int32),
            jax.ShapeDtypeStruct((B * N, O), jnp.float32),
            jax.ShapeDtypeStruct((B * N, GP), jnp.float32),
        ],
        scratch_shapes=[pltpu.VMEM((RB, N), jnp.float32)],
        interpret=interpret,
    )(x, w, bias2d)


def _gather_body(g_hbm, idx_hbm, mx_hbm, mn_hbm, s1_hbm, s2_hbm,
                 idx_v, rows_v, mx_v, mn_v, s1_v, s2_v, sem):
    wid = lax.axis_index("s") * NC + lax.axis_index("c")
    base_pt = wid * PW

    def chunk_body(ci, _):
        pt0 = base_pt + ci * CP
        pltpu.sync_copy(idx_hbm.at[pl.ds(pt0 * K, CP * K)], idx_v)
        pltpu.async_copy(g_hbm.at[idx_v], rows_v, sem).wait()

        def pt_body(p, _):
            for j in range(O // 16):
                sl = pl.ds(j * 16, 16)
                r = rows_v[p * K, sl]
                mx, mn, s1, s2 = r, r, r, r * r
                for k in range(1, K):
                    r = rows_v[p * K + k, sl]
                    mx = jnp.maximum(mx, r)
                    mn = jnp.minimum(mn, r)
                    s1 = s1 + r
                    s2 = s2 + r * r
                mx_v[p, sl] = mx
                mn_v[p, sl] = mn
                s1_v[p, sl] = s1
                s2_v[p, sl] = s2
            return 0

        lax.fori_loop(0, CP, pt_body, 0)
        pltpu.sync_copy(mx_v, mx_hbm.at[pl.ds(pt0, CP)])
        pltpu.sync_copy(mn_v, mn_hbm.at[pl.ds(pt0, CP)])
        pltpu.sync_copy(s1_v, s1_hbm.at[pl.ds(pt0, CP)])
        pltpu.sync_copy(s2_v, s2_hbm.at[pl.ds(pt0, CP)])
        return 0

    lax.fori_loop(0, NCHUNK, chunk_body, 0)


def _gather_call(g, idx_flat):
    f32 = jnp.float32
    run = pl.kernel(
        _gather_body,
        out_type=[jax.ShapeDtypeStruct((B * N, O), f32) for _ in range(4)],
        mesh=plsc.VectorSubcoreMesh(core_axis_name="c", subcore_axis_name="s"),
        scratch_types=[
            pltpu.VMEM((CP * K,), jnp.int32),
            pltpu.VMEM((CP * K, GP), f32),
            pltpu.VMEM((CP, O), f32),
            pltpu.VMEM((CP, O), f32),
            pltpu.VMEM((CP, O), f32),
            pltpu.VMEM((CP, O), f32),
            pltpu.SemaphoreType.DMA,
        ],
    )
    return run(g, idx_flat)


def _stats_body(c_ref, s1_ref, s2_ref, out_ref):
    pid = pl.program_id(0)
    c = c_ref[...]
    s1 = s1_ref[...]
    s2 = s2_ref[...]
    ps = jnp.sum(K * c + s1, axis=0, keepdims=True)
    psq = jnp.sum(K * (c * c) + 2.0 * (c * s1) + s2, axis=0, keepdims=True)
    blk = jnp.concatenate([ps, psq], axis=0)

    @pl.when(pid == 0)
    def _():
        out_ref[...] = blk

    @pl.when(pid != 0)
    def _():
        out_ref[...] = out_ref[...] + blk


def _stats_call(c, s1, s2, interpret=False):
    grid = 16
    rows = (B * N) // grid
    return pl.pallas_call(
        _stats_body,
        grid=(grid,),
        in_specs=[pl.BlockSpec((rows, O), lambda i: (i, 0))] * 3,
        out_specs=pl.BlockSpec((2, O), lambda i: (0, 0)),
        out_shape=jax.ShapeDtypeStruct((2, O), jnp.float32),
        interpret=interpret,
    )(c, s1, s2)


def _final_body(c_ref, mx_ref, mn_ref, stats_ref, gamma_ref, beta_ref, out_ref):
    cnt = float(B * N * K)
    mean = stats_ref[0:1, :] * (1.0 / cnt)
    ex2 = stats_ref[1:2, :] * (1.0 / cnt)
    var = ex2 - mean * mean
    rstd = lax.rsqrt(var + 1e-5)
    gamma = gamma_ref[...]
    beta = beta_ref[...]
    gsel = jnp.where(gamma >= 0.0, mx_ref[...], mn_ref[...])
    z = gamma * ((c_ref[...] + gsel) - mean) * rstd + beta
    z = jnp.where(z > 0, z, 0.2 * z)
    out_ref[0] = z.T


FB = 512


def _final_call(c, mx, mn, stats, gamma2d, beta2d, interpret=False):
    nfb = N // FB
    return pl.pallas_call(
        _final_body,
        grid=(B, nfb),
        in_specs=[
            pl.BlockSpec((FB, O), lambda b, i: (b * nfb + i, 0)),
            pl.BlockSpec((FB, O), lambda b, i: (b * nfb + i, 0)),
            pl.BlockSpec((FB, O), lambda b, i: (b * nfb + i, 0)),
            pl.BlockSpec((2, O), lambda b, i: (0, 0)),
            pl.BlockSpec((1, O), lambda b, i: (0, 0)),
            pl.BlockSpec((1, O), lambda b, i: (0, 0)),
        ],
        out_specs=pl.BlockSpec((1, O, FB), lambda b, i: (b, 0, i)),
        out_shape=jax.ShapeDtypeStruct((B, O, N), jnp.float32),
        interpret=interpret,
    )(c, mx, mn, stats, gamma2d, beta2d)


def kernel(x, W, b, gamma, beta):
    idx, c, g = _knn_call(x, W, b.reshape(1, O))
    mx, mn, s1, s2 = _gather_call(g, idx.reshape(B * N * K))
    stats = _stats_call(c, s1, s2)
    return _final_call(c, mx, mn, stats, gamma.reshape(1, O), beta.reshape(1, O))
